# baseline (speedup 1.0000x reference)
import functools

import jax
import jax.numpy as jnp
from jax import lax
from jax.experimental import pallas as pl
from jax.experimental.pallas import tpu as pltpu
from jax.experimental.pallas import tpu_sc as plsc

BATCH = 16384
DIM = 128
NROWS = 8
CHUNK = 128


def _body(types_hbm, labels_hbm, table_hbm, out_hbm,
          t_v, l_v, table_v, stab, rows_v, gsem, osem, *, bpw):
    info = plsc.get_sparse_core_info()
    nc, lanes = info.num_cores, info.num_lanes
    nchunk = bpw // CHUNK

    sid = lax.axis_index("s")
    wid = sid * nc + lax.axis_index("c")
    base = wid * bpw
    row_off = sid * NROWS

    pltpu.sync_copy(table_hbm, table_v)
    pltpu.sync_copy(table_v, stab.at[pl.ds(row_off, NROWS)])

    pltpu.sync_copy(types_hbm.at[pl.ds(base, bpw)], t_v)
    pltpu.sync_copy(labels_hbm.at[pl.ds(base, bpw)], l_v)

    gpc = CHUNK // lanes
    gathers = []
    for g in range(bpw // lanes):
        s = pl.ds(g * lanes, lanes)
        ids = t_v[s] * 2 + l_v[s] + row_off
        gathers.append(
            pltpu.async_copy(stab.at[ids],
                             rows_v.at[pl.ds(g * lanes, lanes)],
                             gsem.at[g // gpc]))
    stores = []
    for j in range(nchunk):
        for cp in gathers[j * gpc:(j + 1) * gpc]:
            cp.wait()
        stores.append(
            pltpu.async_copy(rows_v.at[pl.ds(j * CHUNK, CHUNK)],
                             out_hbm.at[pl.ds(base + j * CHUNK, CHUNK)],
                             osem))
    for s_ in stores:
        s_.wait()


def _sc_call(types, labels, table):
    info = plsc.get_sparse_core_info()
    nw = info.num_cores * info.num_subcores
    n = types.shape[0]
    bpw = n // nw
    nchunk = bpw // CHUNK
    mesh = plsc.VectorSubcoreMesh(core_axis_name="c", subcore_axis_name="s")
    f = functools.partial(
        pl.kernel,
        mesh=mesh,
        out_type=jax.ShapeDtypeStruct((n, DIM), jnp.float32),
        scratch_types=[
            pltpu.VMEM((bpw,), jnp.int32),
            pltpu.VMEM((bpw,), jnp.int32),
            pltpu.VMEM((NROWS, DIM), jnp.float32),
            pltpu.VMEM_SHARED((16 * NROWS, DIM), jnp.float32),
            pltpu.VMEM((bpw, DIM), jnp.float32),
            pltpu.SemaphoreType.DMA((nchunk,)),
            pltpu.SemaphoreType.DMA,
        ],
    )(functools.partial(_body, bpw=bpw))
    return f(types, labels, table)


TCB = 1024


def _tc_body(t_ref, l_ref, tab_ref, o_ref):
    idx = t_ref[0] * 2 + l_ref[0]
    oh = (lax.broadcasted_iota(jnp.int32, (NROWS, TCB), 0) == idx
          ).astype(jnp.float32)
    o_ref[...] = lax.dot_general(
        oh, tab_ref[...], (((0,), (0,)), ((), ())),
        precision=lax.Precision.HIGHEST,
        preferred_element_type=jnp.float32)


def _tc_call(types, labels, table):
    n = types.shape[0]
    nb = n // TCB
    return pl.pallas_call(
        _tc_body,
        grid=(nb,),
        in_specs=[
            pl.BlockSpec((1, 1, TCB), lambda i: (i, 0, 0)),
            pl.BlockSpec((1, 1, TCB), lambda i: (i, 0, 0)),
            pl.BlockSpec((NROWS, DIM), lambda i: (0, 0)),
        ],
        out_specs=pl.BlockSpec((TCB, DIM), lambda i: (i, 0)),
        out_shape=jax.ShapeDtypeStruct((n, DIM), jnp.float32),
    )(types.reshape(nb, 1, TCB), labels.reshape(nb, 1, TCB), table)


SC_FRAC = 2


def kernel(interaction_types, labels, embedding_weight):
    t = interaction_types.astype(jnp.int32)
    l = labels.astype(jnp.int32)
    nsc = BATCH // SC_FRAC
    out_sc = _sc_call(t[:nsc], l[:nsc], embedding_weight)
    out_tc = _tc_call(t[nsc:], l[nsc:], embedding_weight)
    return jnp.concatenate([out_sc, out_tc], axis=0)


# speedup vs baseline: 1.3307x; 1.3307x over previous
import functools

import jax
import jax.numpy as jnp
from jax import lax
from jax.experimental import pallas as pl
from jax.experimental.pallas import tpu as pltpu
from jax.experimental.pallas import tpu_sc as plsc

BATCH = 16384
DIM = 128
NROWS = 8
CHUNK = 64


def _body(types_hbm, labels_hbm, table_hbm, out_hbm,
          t_v, l_v, table_v, stab, rows_v, gsem, osem, *, bpw):
    info = plsc.get_sparse_core_info()
    nc, lanes = info.num_cores, info.num_lanes
    nchunk = bpw // CHUNK

    sid = lax.axis_index("s")
    wid = sid * nc + lax.axis_index("c")
    base = wid * bpw
    row_off = sid * NROWS

    pltpu.sync_copy(table_hbm, table_v)
    pltpu.sync_copy(table_v, stab.at[pl.ds(row_off, NROWS)])

    pltpu.sync_copy(types_hbm.at[pl.ds(base, bpw)], t_v)
    pltpu.sync_copy(labels_hbm.at[pl.ds(base, bpw)], l_v)

    gpc = CHUNK // lanes
    gathers = []
    for g in range(bpw // lanes):
        s = pl.ds(g * lanes, lanes)
        ids = t_v[s] * 2 + l_v[s] + row_off
        gathers.append(
            pltpu.async_copy(stab.at[ids],
                             rows_v.at[pl.ds(g * lanes, lanes)],
                             gsem.at[g // gpc]))
    stores = []
    for j in range(nchunk):
        for cp in gathers[j * gpc:(j + 1) * gpc]:
            cp.wait()
        stores.append(
            pltpu.async_copy(rows_v.at[pl.ds(j * CHUNK, CHUNK)],
                             out_hbm.at[pl.ds(base + j * CHUNK, CHUNK)],
                             osem))
    for s_ in stores:
        s_.wait()


def _sc_call(types, labels, table):
    info = plsc.get_sparse_core_info()
    nw = info.num_cores * info.num_subcores
    n = types.shape[0]
    bpw = n // nw
    nchunk = bpw // CHUNK
    mesh = plsc.VectorSubcoreMesh(core_axis_name="c", subcore_axis_name="s")
    f = functools.partial(
        pl.kernel,
        mesh=mesh,
        out_type=jax.ShapeDtypeStruct((n, DIM), jnp.float32),
        scratch_types=[
            pltpu.VMEM((bpw,), jnp.int32),
            pltpu.VMEM((bpw,), jnp.int32),
            pltpu.VMEM((NROWS, DIM), jnp.float32),
            pltpu.VMEM_SHARED((16 * NROWS, DIM), jnp.float32),
            pltpu.VMEM((bpw, DIM), jnp.float32),
            pltpu.SemaphoreType.DMA((nchunk,)),
            pltpu.SemaphoreType.DMA,
        ],
    )(functools.partial(_body, bpw=bpw))
    return f(types, labels, table)


def kernel(interaction_types, labels, embedding_weight):
    return _sc_call(interaction_types.astype(jnp.int32),
                    labels.astype(jnp.int32),
                    embedding_weight)
